# baseline (speedup 1.0000x reference)
import functools

import jax
import jax.numpy as jnp
from jax import lax
from jax.experimental import pallas as pl
from jax.experimental.pallas import tpu as pltpu
from jax.experimental.pallas import tpu_sc as plsc

DIM = 32
EPS = 1e-08
NW = 32
CHUNK = 256
SUB = CHUNK // 128
NBUF = 3


def _rsqrt16(x):
    i = lax.bitcast_convert_type(x, jnp.int32)
    y = lax.bitcast_convert_type(
        jnp.int32(0x5F3759DF) - lax.shift_right_logical(i, 1), jnp.float32
    )
    for _ in range(3):
        y = y * (1.5 - 0.5 * x * y * y)
    return jnp.where(x < jnp.float32(EPS * EPS), jnp.float32(1.0 / EPS), y)


def _make_kernel(n_rows):
    bpw = n_rows // NW
    n_chunks = bpw // CHUNK
    mesh = plsc.VectorSubcoreMesh(core_axis_name="c", subcore_axis_name="s")

    @functools.partial(
        pl.kernel,
        out_type=(
            jax.ShapeDtypeStruct((n_rows, DIM), jnp.float32),
            jax.ShapeDtypeStruct((n_rows, DIM), jnp.float32),
            jax.ShapeDtypeStruct((n_rows, 2 * DIM), jnp.float32),
        ),
        mesh=mesh,
        scratch_types=[
            pltpu.VMEM((NBUF, SUB, 128), jnp.int32),
            pltpu.VMEM((NBUF, CHUNK, DIM), jnp.float32),
            pltpu.VMEM((NBUF, CHUNK, DIM), jnp.float32),
            pltpu.VMEM((NBUF, CHUNK, 2 * DIM), jnp.float32),
            pltpu.SemaphoreType.DMA((NBUF,)),
            pltpu.SemaphoreType.DMA((NBUF,)),
        ],
        compiler_params=pltpu.CompilerParams(
            needs_layout_passes=False, use_tc_tiling_on_sc=False
        ),
    )
    def kern(ids_hbm, base_hbm, ctx_hbm, qb_hbm, qc_hbm, qt_hbm,
             idx_v, base_v, ctx_v, tot_v, gsem, osem):
        wid = lax.axis_index("s") * 2 + lax.axis_index("c")
        row0 = wid * bpw
        lanes = lax.iota(jnp.int32, 16)

        def issue_gathers(g, slot):
            start = row0 + g * CHUNK
            pltpu.sync_copy(ids_hbm.at[start // CHUNK], idx_v.at[slot])
            for j in range(SUB):
                pltpu.async_copy(
                    base_hbm.at[idx_v.at[slot, j]],
                    base_v.at[slot, pl.ds(j * 128, 128)], gsem.at[slot])
                pltpu.async_copy(
                    ctx_hbm.at[idx_v.at[slot, j]],
                    ctx_v.at[slot, pl.ds(j * 128, 128)], gsem.at[slot])

        def wait_gathers(g, slot):
            for j in range(SUB):
                pltpu.make_async_copy(
                    base_hbm.at[idx_v.at[slot, j]],
                    base_v.at[slot, pl.ds(j * 128, 128)], gsem.at[slot]).wait()
                pltpu.make_async_copy(
                    ctx_hbm.at[idx_v.at[slot, j]],
                    ctx_v.at[slot, pl.ds(j * 128, 128)], gsem.at[slot]).wait()

        def issue_outputs(g, slot):
            start = row0 + g * CHUNK
            pltpu.async_copy(
                base_v.at[slot], qb_hbm.at[pl.ds(start, CHUNK)], osem.at[slot])
            pltpu.async_copy(
                ctx_v.at[slot], qc_hbm.at[pl.ds(start, CHUNK)], osem.at[slot])
            pltpu.async_copy(
                tot_v.at[slot], qt_hbm.at[pl.ds(start, CHUNK)], osem.at[slot])

        def drain_outputs(g, slot):
            start = row0 + g * CHUNK
            pltpu.make_async_copy(
                base_v.at[slot], qb_hbm.at[pl.ds(start, CHUNK)],
                osem.at[slot]).wait()
            pltpu.make_async_copy(
                ctx_v.at[slot], qc_hbm.at[pl.ds(start, CHUNK)],
                osem.at[slot]).wait()
            pltpu.make_async_copy(
                tot_v.at[slot], qt_hbm.at[pl.ds(start, CHUNK)],
                osem.at[slot]).wait()

        def compute(slot):
            bv = base_v.at[slot]
            cv = ctx_v.at[slot]
            tv = tot_v.at[slot]

            def blk_body(blk, _):
                rows = blk * 16 + lanes
                acc = jnp.zeros((16,), jnp.float32)
                for d in range(DIM):
                    col = jnp.full((16,), d, jnp.int32)
                    vb = plsc.load_gather(bv, [rows, col])
                    vc = plsc.load_gather(cv, [rows, col])
                    acc = acc + vb * vb + vc * vc
                inv = _rsqrt16(acc)
                for d in range(DIM):
                    col = jnp.full((16,), d, jnp.int32)
                    col2 = jnp.full((16,), DIM + d, jnp.int32)
                    vb = plsc.load_gather(bv, [rows, col])
                    vc = plsc.load_gather(cv, [rows, col])
                    plsc.store_scatter(tv, [rows, col], vb * inv)
                    plsc.store_scatter(tv, [rows, col2], vc * inv)
                return 0

            lax.fori_loop(0, CHUNK // 16, blk_body, 0)

        issue_gathers(0, 0)

        def chunk_body(g, _):
            slot = lax.rem(g, NBUF)
            nslot = lax.rem(g + 1, NBUF)

            @pl.when(g >= 2)
            def _():
                drain_outputs(g - 2, nslot)

            @pl.when(g + 1 < n_chunks)
            def _():
                issue_gathers(g + 1, nslot)

            wait_gathers(g, slot)
            compute(slot)
            issue_outputs(g, slot)
            return 0

        lax.fori_loop(0, n_chunks, chunk_body, 0)
        drain_outputs(n_chunks - 2, lax.rem(n_chunks - 2, NBUF))
        drain_outputs(n_chunks - 1, lax.rem(n_chunks - 1, NBUF))

    return kern


def kernel(concept_ids, base_table, context_table):
    b, s = concept_ids.shape
    n = b * s
    ids3d = concept_ids.reshape(n // CHUNK, SUB, 128).astype(jnp.int32)
    qb, qc, qt = _make_kernel(n)(ids3d, base_table, context_table)
    return (
        qb.reshape(b, s, DIM),
        qc.reshape(b, s, DIM),
        qt.reshape(b, s, 2 * DIM),
    )


# speedup vs baseline: 1.6243x; 1.6243x over previous
import functools

import jax
import jax.numpy as jnp
from jax import lax
from jax.experimental import pallas as pl
from jax.experimental.pallas import tpu as pltpu
from jax.experimental.pallas import tpu_sc as plsc

DIM = 32
EPS = 1e-08
NW = 32
CHUNK = 256
SUB = CHUNK // 128
NBUF = 3


def _rsqrt16(x):
    i = lax.bitcast_convert_type(x, jnp.int32)
    y = lax.bitcast_convert_type(
        jnp.int32(0x5F3759DF) - lax.shift_right_logical(i, 1), jnp.float32
    )
    for _ in range(3):
        y = y * (1.5 - 0.5 * x * y * y)
    return jnp.where(x < jnp.float32(EPS * EPS), jnp.float32(1.0 / EPS), y)


def _make_kernel(n_rows):
    bpw = n_rows // NW
    n_chunks = bpw // CHUNK
    mesh = plsc.VectorSubcoreMesh(core_axis_name="c", subcore_axis_name="s")

    @functools.partial(
        pl.kernel,
        out_type=(
            jax.ShapeDtypeStruct((n_rows, DIM), jnp.float32),
            jax.ShapeDtypeStruct((n_rows, DIM), jnp.float32),
            jax.ShapeDtypeStruct((n_rows, 2 * DIM), jnp.float32),
        ),
        mesh=mesh,
        scratch_types=[
            pltpu.VMEM((NBUF, SUB, 128), jnp.int32),
            pltpu.VMEM((NBUF, CHUNK, DIM), jnp.float32),
            pltpu.VMEM((NBUF, CHUNK, DIM), jnp.float32),
            pltpu.VMEM((NBUF, CHUNK, 2 * DIM), jnp.float32),
            pltpu.SemaphoreType.DMA((NBUF,)),
            pltpu.SemaphoreType.DMA((NBUF,)),
        ],
        compiler_params=pltpu.CompilerParams(
            needs_layout_passes=False, use_tc_tiling_on_sc=False
        ),
    )
    def kern(ids_hbm, base_hbm, ctx_hbm, qb_hbm, qc_hbm, qt_hbm,
             idx_v, base_v, ctx_v, tot_v, gsem, osem):
        wid = lax.axis_index("s") * 2 + lax.axis_index("c")
        row0 = wid * bpw
        lanes = lax.iota(jnp.int32, 16)

        def issue_gathers(g, slot):
            start = row0 + g * CHUNK
            pltpu.sync_copy(ids_hbm.at[start // CHUNK], idx_v.at[slot])
            for j in range(SUB):
                pltpu.async_copy(
                    base_hbm.at[idx_v.at[slot, j]],
                    base_v.at[slot, pl.ds(j * 128, 128)], gsem.at[slot])
                pltpu.async_copy(
                    ctx_hbm.at[idx_v.at[slot, j]],
                    ctx_v.at[slot, pl.ds(j * 128, 128)], gsem.at[slot])

        def wait_gathers(g, slot):
            for j in range(SUB):
                pltpu.make_async_copy(
                    base_hbm.at[idx_v.at[slot, j]],
                    base_v.at[slot, pl.ds(j * 128, 128)], gsem.at[slot]).wait()
                pltpu.make_async_copy(
                    ctx_hbm.at[idx_v.at[slot, j]],
                    ctx_v.at[slot, pl.ds(j * 128, 128)], gsem.at[slot]).wait()

        def issue_outputs(g, slot):
            start = row0 + g * CHUNK
            pltpu.async_copy(
                base_v.at[slot], qb_hbm.at[pl.ds(start, CHUNK)], osem.at[slot])
            pltpu.async_copy(
                ctx_v.at[slot], qc_hbm.at[pl.ds(start, CHUNK)], osem.at[slot])
            pltpu.async_copy(
                tot_v.at[slot], qt_hbm.at[pl.ds(start, CHUNK)], osem.at[slot])

        def drain_outputs(g, slot):
            start = row0 + g * CHUNK
            pltpu.make_async_copy(
                base_v.at[slot], qb_hbm.at[pl.ds(start, CHUNK)],
                osem.at[slot]).wait()
            pltpu.make_async_copy(
                ctx_v.at[slot], qc_hbm.at[pl.ds(start, CHUNK)],
                osem.at[slot]).wait()
            pltpu.make_async_copy(
                tot_v.at[slot], qt_hbm.at[pl.ds(start, CHUNK)],
                osem.at[slot]).wait()

        def compute(slot):
            def row_body(r, _):
                vb0 = base_v[slot, r, pl.ds(0, 16)]
                vb1 = base_v[slot, r, pl.ds(16, 16)]
                vc0 = ctx_v[slot, r, pl.ds(0, 16)]
                vc1 = ctx_v[slot, r, pl.ds(16, 16)]
                s = vb0 * vb0 + vb1 * vb1 + vc0 * vc0 + vc1 * vc1
                inv = _rsqrt16(jnp.full((16,), jnp.sum(s), jnp.float32))
                tot_v[slot, r, pl.ds(0, 16)] = vb0 * inv
                tot_v[slot, r, pl.ds(16, 16)] = vb1 * inv
                tot_v[slot, r, pl.ds(32, 16)] = vc0 * inv
                tot_v[slot, r, pl.ds(48, 16)] = vc1 * inv
                return 0

            lax.fori_loop(0, CHUNK, row_body, 0)

        issue_gathers(0, 0)

        def chunk_body(g, _):
            slot = lax.rem(g, NBUF)
            nslot = lax.rem(g + 1, NBUF)

            @pl.when(g >= 2)
            def _():
                drain_outputs(g - 2, nslot)

            @pl.when(g + 1 < n_chunks)
            def _():
                issue_gathers(g + 1, nslot)

            wait_gathers(g, slot)
            compute(slot)
            issue_outputs(g, slot)
            return 0

        lax.fori_loop(0, n_chunks, chunk_body, 0)
        drain_outputs(n_chunks - 2, lax.rem(n_chunks - 2, NBUF))
        drain_outputs(n_chunks - 1, lax.rem(n_chunks - 1, NBUF))

    return kern


def kernel(concept_ids, base_table, context_table):
    b, s = concept_ids.shape
    n = b * s
    ids3d = concept_ids.reshape(n // CHUNK, SUB, 128).astype(jnp.int32)
    qb, qc, qt = _make_kernel(n)(ids3d, base_table, context_table)
    return (
        qb.reshape(b, s, DIM),
        qc.reshape(b, s, DIM),
        qt.reshape(b, s, 2 * DIM),
    )
